# baseline (speedup 1.0000x reference)
import functools

import jax
import jax.numpy as jnp
from jax import lax
from jax.experimental import pallas as pl
from jax.experimental.pallas import tpu as pltpu
from jax.experimental.pallas import tpu_sc as plsc

H = 8
C = 16
D = 128

NC = 2
NS = 16
NW = NC * NS
K = 112
ZB = 80


def _proj_body(x_ref, w_ref, asrc_ref, adst_ref, wedge_ref, aedge_ref,
               xw_ref, as_ref, ad_ref, s_ref):
    xw = jnp.dot(x_ref[...], w_ref[...], preferred_element_type=jnp.float32)
    xw_ref[...] = xw
    jj = lax.broadcasted_iota(jnp.int32, (D, H), 0) // C
    hh = lax.broadcasted_iota(jnp.int32, (D, H), 1)
    sel = (jj == hh).astype(jnp.float32)
    as_ref[...] = jnp.dot(xw * asrc_ref[...], sel,
                          preferred_element_type=jnp.float32)
    ad_ref[...] = jnp.dot(xw * adst_ref[...], sel,
                          preferred_element_type=jnp.float32)
    s_ref[...] = jnp.dot(wedge_ref[...] * aedge_ref[...], sel,
                         preferred_element_type=jnp.float32)


def _project(x, w, asrc_flat, adst_flat, wedge, aedge_flat, n):
    bn = 1000
    grid = (n // bn,)
    return pl.pallas_call(
        _proj_body,
        grid=grid,
        in_specs=[
            pl.BlockSpec((bn, D), lambda i: (i, 0)),
            pl.BlockSpec((D, D), lambda i: (0, 0)),
            pl.BlockSpec((1, D), lambda i: (0, 0)),
            pl.BlockSpec((1, D), lambda i: (0, 0)),
            pl.BlockSpec((1, D), lambda i: (0, 0)),
            pl.BlockSpec((1, D), lambda i: (0, 0)),
        ],
        out_specs=[
            pl.BlockSpec((bn, D), lambda i: (i, 0)),
            pl.BlockSpec((bn, H), lambda i: (i, 0)),
            pl.BlockSpec((bn, H), lambda i: (i, 0)),
            pl.BlockSpec((1, H), lambda i: (0, 0)),
        ],
        out_shape=[
            jax.ShapeDtypeStruct((n, D), jnp.float32),
            jax.ShapeDtypeStruct((n, H), jnp.float32),
            jax.ShapeDtypeStruct((n, H), jnp.float32),
            jax.ShapeDtypeStruct((1, H), jnp.float32),
        ],
    )(x, w, asrc_flat, adst_flat, wedge, aedge_flat)


def _make_sc_edge_pass(n, n_pad, e):
    total_chunks = -(-e // K)
    chunks_per_tile = -(-total_chunks // NW)
    chunks_per_tile += chunks_per_tile % 2
    rem = e % K
    e_floor = (e // K) * K
    rows_per_tile = n_pad // NS
    mesh = plsc.VectorSubcoreMesh(core_axis_name="c", subcore_axis_name="s")

    def body(ei_hbm, attr_hbm, asrc_hbm, adst_hbm, xw_hbm, svec_hbm,
             outp_hbm, denp_hbm,
             src_v, dst_v, attr_v, as_v, ad_v, xw_v, p_v, svec_v, stage_v,
             out_sh, den_sh, sem_ld, sem_g, sem_sc):
        cid = lax.axis_index("c")
        sid = lax.axis_index("s")
        wid = cid * NS + sid
        nch = chunks_per_tile

        pltpu.sync_copy(svec_hbm, svec_v)

        zeros16 = jnp.zeros((16,), jnp.float32)

        def zero_row(i, _):
            for j in range(D // 16):
                xw_v[0][i, pl.ds(j * 16, 16)] = zeros16
            p_v[0][i, pl.ds(0, 16)] = zeros16
            return 0

        lax.fori_loop(0, K, zero_row, 0)

        rbase = sid * rows_per_tile

        def zero_stripe(j, _):
            pltpu.sync_copy(xw_v[0].at[pl.ds(0, ZB)],
                            out_sh.at[pl.ds(rbase + j * ZB, ZB)])
            pltpu.sync_copy(p_v[0].at[pl.ds(0, ZB)],
                            den_sh.at[pl.ds(rbase + j * ZB, ZB)])
            return 0

        lax.fori_loop(0, rows_per_tile // ZB, zero_stripe, 0)
        plsc.subcore_barrier()

        lanes = lax.iota(jnp.int32, 16)
        ones16 = jnp.full((16,), 1.0, jnp.float32)
        col8 = jnp.full((16,), 8, jnp.int32)
        col9 = jnp.full((16,), 9, jnp.int32)

        def chunk_base(k):
            return (k * NW + wid) * K

        def load_idx(k, b):
            ebase = chunk_base(k)
            pltpu.async_copy(ei_hbm.at[pl.ds(ebase, K)], src_v[b],
                             sem_ld[0])
            pltpu.async_copy(ei_hbm.at[pl.ds(e + ebase, K)], dst_v[b],
                             sem_ld[0])
            pltpu.async_copy(attr_hbm.at[pl.ds(ebase, K)], attr_v[b],
                             sem_ld[0])

        def wait_idx(b):
            pltpu.make_async_copy(ei_hbm.at[pl.ds(0, K)], src_v[b],
                                  sem_ld[0]).wait()
            pltpu.make_async_copy(ei_hbm.at[pl.ds(0, K)], dst_v[b],
                                  sem_ld[0]).wait()
            pltpu.make_async_copy(attr_hbm.at[pl.ds(0, K)], attr_v[b],
                                  sem_ld[0]).wait()

        def load_stage(k):
            ebase = chunk_base(k)

            @pl.when(ebase + K <= e)
            def _full():
                pltpu.async_copy(ei_hbm.at[pl.ds(ebase, K)], stage_v[0],
                                 sem_ld[1])
                pltpu.async_copy(ei_hbm.at[pl.ds(e + ebase, K)], stage_v[1],
                                 sem_ld[1])
                pltpu.async_copy(attr_hbm.at[pl.ds(ebase, K)], stage_v[2],
                                 sem_ld[1])

            if rem:
                @pl.when(ebase == e_floor)
                def _partial():
                    pltpu.async_copy(ei_hbm.at[pl.ds(ebase, rem)],
                                     stage_v[0].at[pl.ds(0, rem)], sem_ld[1])
                    pltpu.async_copy(ei_hbm.at[pl.ds(e + ebase, rem)],
                                     stage_v[1].at[pl.ds(0, rem)], sem_ld[1])
                    pltpu.async_copy(attr_hbm.at[pl.ds(ebase, rem)],
                                     stage_v[2].at[pl.ds(0, rem)], sem_ld[1])

        def unstage(b, k):
            ebase = chunk_base(k)

            @pl.when(ebase + K <= e)
            def _full():
                pltpu.make_async_copy(ei_hbm.at[pl.ds(0, K)], stage_v[0],
                                      sem_ld[1]).wait()
                pltpu.make_async_copy(ei_hbm.at[pl.ds(0, K)], stage_v[1],
                                      sem_ld[1]).wait()
                pltpu.make_async_copy(attr_hbm.at[pl.ds(0, K)], stage_v[2],
                                      sem_ld[1]).wait()

            if rem:
                @pl.when(ebase == e_floor)
                def _partial():
                    pltpu.make_async_copy(ei_hbm.at[pl.ds(0, rem)],
                                          stage_v[0].at[pl.ds(0, rem)],
                                          sem_ld[1]).wait()
                    pltpu.make_async_copy(ei_hbm.at[pl.ds(0, rem)],
                                          stage_v[1].at[pl.ds(0, rem)],
                                          sem_ld[1]).wait()
                    pltpu.make_async_copy(attr_hbm.at[pl.ds(0, rem)],
                                          stage_v[2].at[pl.ds(0, rem)],
                                          sem_ld[1]).wait()

            @plsc.parallel_loop(0, K // 16, 1, unroll=2)
            def cp(g):
                sl = pl.ds(g * 16, 16)
                src_v[b][sl] = stage_v[0][sl]
                dst_v[b][sl] = stage_v[1][sl]
                attr_v[b][sl] = stage_v[2][sl]

        def start_gathers(b):
            pltpu.async_copy(asrc_hbm.at[src_v[b]], as_v[b], sem_g[b])
            pltpu.async_copy(adst_hbm.at[dst_v[b]], ad_v[b], sem_g[b])
            pltpu.async_copy(xw_hbm.at[src_v[b]], xw_v[b], sem_g[b])

        def wait_scatters(b, k):
            @pl.when(chunk_base(k) < e)
            def _():
                pltpu.make_async_copy(xw_v[b], out_sh.at[dst_v[b]],
                                      sem_sc[b]).wait()
                pltpu.make_async_copy(p_v[b], den_sh.at[dst_v[b]],
                                      sem_sc[b]).wait()

        def process(k, b):
            nb = 1 - b
            @pl.when(k + 1 < nch)
            def _prefetch():
                @pl.when(k >= 1)
                def _drain():
                    wait_scatters(nb, k - 1)

                unstage(nb, k + 1)

                @pl.when(chunk_base(k + 1) < e)
                def _gather():
                    start_gathers(nb)

            @pl.when(k + 2 < nch)
            def _stage():
                load_stage(k + 2)

            ebase = chunk_base(k)

            @pl.when(ebase < e)
            def _active():
                pltpu.make_async_copy(asrc_hbm.at[pl.ds(0, K)], as_v[b],
                                      sem_g[b]).wait()
                pltpu.make_async_copy(adst_hbm.at[pl.ds(0, K)], ad_v[b],
                                      sem_g[b]).wait()

                @plsc.parallel_loop(0, K // 16, 1, unroll=2)
                def grp(g):
                    eids = lanes + g * 16
                    valid = (ebase + eids) < e
                    att16 = attr_v[b][pl.ds(g * 16, 16)]
                    s_all = svec_v[pl.ds(0, 16)]
                    for h in range(H):
                        hv = jnp.full((16,), h, jnp.int32)
                        a_s = plsc.load_gather(as_v[b], [eids, hv])
                        a_d = plsc.load_gather(ad_v[b], [eids, hv])
                        al = a_s + a_d + att16 * s_all[h]
                        al = jnp.where(al >= 0.0, al, al * 0.2)
                        p = jnp.where(valid, jnp.exp(al), 0.0)
                        plsc.store_scatter(p_v[b], [eids, hv], p)
                    plsc.store_scatter(p_v[b], [eids, col8],
                                       jnp.where(valid, ones16, 0.0))
                    plsc.store_scatter(p_v[b], [eids, col9], att16)

                pltpu.make_async_copy(xw_hbm.at[pl.ds(0, K)], xw_v[b],
                                      sem_g[b]).wait()

                @plsc.parallel_loop(0, K, 1, unroll=8)
                def edge_mul(i):
                    prow = p_v[b][i, pl.ds(0, 16)]
                    for h in range(H):
                        xw_v[b][i, pl.ds(h * 16, 16)] = (
                            xw_v[b][i, pl.ds(h * 16, 16)] * prow[h])

                pltpu.async_copy(xw_v[b], out_sh.at[dst_v[b]], sem_sc[b],
                                 add=True)
                pltpu.async_copy(p_v[b], den_sh.at[dst_v[b]], sem_sc[b],
                                 add=True)

        load_idx(0, 0)
        wait_idx(0)
        start_gathers(0)
        load_stage(1)

        def pair_body(j, _):
            process(2 * j, 0)
            process(2 * j + 1, 1)
            return 0

        lax.fori_loop(0, nch // 2, pair_body, 0)
        wait_scatters(0, nch - 2)
        wait_scatters(1, nch - 1)
        plsc.subcore_barrier()

        pltpu.sync_copy(out_sh.at[pl.ds(rbase, rows_per_tile)],
                        outp_hbm.at[cid, pl.ds(rbase, rows_per_tile)])
        pltpu.sync_copy(den_sh.at[pl.ds(rbase, rows_per_tile)],
                        denp_hbm.at[cid, pl.ds(rbase, rows_per_tile)])

    return pl.kernel(
        body,
        out_type=[
            jax.ShapeDtypeStruct((NC, n_pad, D), jnp.float32),
            jax.ShapeDtypeStruct((NC, n_pad, 16), jnp.float32),
        ],
        mesh=mesh,
        compiler_params=pltpu.CompilerParams(
            needs_layout_passes=False, use_tc_tiling_on_sc=False),
        scratch_types=[
            (pltpu.VMEM((K,), jnp.int32), pltpu.VMEM((K,), jnp.int32)),
            (pltpu.VMEM((K,), jnp.int32), pltpu.VMEM((K,), jnp.int32)),
            (pltpu.VMEM((K,), jnp.float32), pltpu.VMEM((K,), jnp.float32)),
            (pltpu.VMEM((K, H), jnp.float32), pltpu.VMEM((K, H), jnp.float32)),
            (pltpu.VMEM((K, H), jnp.float32), pltpu.VMEM((K, H), jnp.float32)),
            (pltpu.VMEM((K, D), jnp.float32), pltpu.VMEM((K, D), jnp.float32)),
            (pltpu.VMEM((K, 16), jnp.float32), pltpu.VMEM((K, 16), jnp.float32)),
            pltpu.VMEM((16,), jnp.float32),
            (pltpu.VMEM((K,), jnp.int32), pltpu.VMEM((K,), jnp.int32),
             pltpu.VMEM((K,), jnp.float32)),
            pltpu.VMEM_SHARED((n_pad, D), jnp.float32),
            pltpu.VMEM_SHARED((n_pad, 16), jnp.float32),
            (pltpu.SemaphoreType.DMA, pltpu.SemaphoreType.DMA),
            (pltpu.SemaphoreType.DMA, pltpu.SemaphoreType.DMA),
            (pltpu.SemaphoreType.DMA, pltpu.SemaphoreType.DMA),
        ],
    )


def _final_body(outp_ref, denp_ref, xw_ref, as_ref, ad_ref, s_ref,
                bias_ref, gamma_ref, beta_ref, out_ref):
    den = denp_ref[0] + denp_ref[1]
    acc = outp_ref[0] + outp_ref[1]
    denh = den[:, 0:H]
    deg = den[:, H:H + 1]
    asum = den[:, H + 1:H + 2]
    loop_attr = asum / jnp.maximum(deg, 1.0)
    al = as_ref[...] + ad_ref[...] + loop_attr * s_ref[...]
    al = jnp.where(al >= 0.0, al, al * 0.2)
    p_loop = jnp.exp(al)
    denom = denh + p_loop
    hh = lax.broadcasted_iota(jnp.int32, (H, D), 0)
    jj = lax.broadcasted_iota(jnp.int32, (H, D), 1) // C
    sel = (hh == jj).astype(jnp.float32)
    acc = acc + jnp.dot(p_loop, sel, preferred_element_type=jnp.float32) * xw_ref[...]
    acc = acc / (jnp.dot(denom, sel, preferred_element_type=jnp.float32) + 1e-16)
    acc = (acc + bias_ref[...]) * (gamma_ref[...] * (1.0 / jnp.sqrt(1.0 + 1e-5))) + beta_ref[...]
    out_ref[...] = jnp.where(acc > 0.0, acc, jnp.exp(acc) - 1.0)


def _finalize(outp, denp, xw, a_src, a_dst, s_row, bias, gamma, beta, n):
    bn = 1000
    grid = (n // bn,)
    return pl.pallas_call(
        _final_body,
        grid=grid,
        in_specs=[
            pl.BlockSpec((NC, bn, D), lambda i: (0, i, 0)),
            pl.BlockSpec((NC, bn, 16), lambda i: (0, i, 0)),
            pl.BlockSpec((bn, D), lambda i: (i, 0)),
            pl.BlockSpec((bn, H), lambda i: (i, 0)),
            pl.BlockSpec((bn, H), lambda i: (i, 0)),
            pl.BlockSpec((1, H), lambda i: (0, 0)),
            pl.BlockSpec((1, D), lambda i: (0, 0)),
            pl.BlockSpec((1, D), lambda i: (0, 0)),
            pl.BlockSpec((1, D), lambda i: (0, 0)),
        ],
        out_specs=pl.BlockSpec((bn, D), lambda i: (i, 0)),
        out_shape=jax.ShapeDtypeStruct((n, D), jnp.float32),
    )(outp, denp, xw, a_src, a_dst, s_row, bias, gamma, beta)


@jax.jit
def kernel(x, edge_index, edge_attr, W, att_src, att_dst, W_edge, att_edge,
           bias, bn_gamma, bn_beta):
    n = x.shape[0]
    e = edge_index.shape[1]

    ei = edge_index.astype(jnp.int32)

    asrc_flat = att_src.reshape(1, D)
    adst_flat = att_dst.reshape(1, D)
    aedge_flat = att_edge.reshape(1, D)

    xw, a_src, a_dst, s_row = _project(x, W, asrc_flat, adst_flat,
                                       W_edge, aedge_flat, n)

    s16 = jnp.concatenate([s_row.reshape(H), jnp.zeros((16 - H,), jnp.float32)])
    n_pad = ((n + NS * ZB - 1) // (NS * ZB)) * (NS * ZB)
    outp, denp = _make_sc_edge_pass(n, n_pad, e)(
        ei.reshape(2 * e), edge_attr.reshape(e), a_src, a_dst, xw, s16)

    return _finalize(outp, denp, xw, a_src, a_dst, s_row,
                     bias.reshape(1, D), bn_gamma.reshape(1, D),
                     bn_beta.reshape(1, D), n)


# speedup vs baseline: 1.1728x; 1.1728x over previous
import functools

import jax
import jax.numpy as jnp
from jax import lax
from jax.experimental import pallas as pl
from jax.experimental.pallas import tpu as pltpu
from jax.experimental.pallas import tpu_sc as plsc

H = 8
C = 16
D = 128

NC = 2
NS = 16
NW = NC * NS
K = 112
ZB = 80


def _proj_body(x_ref, w_ref, asrc_ref, adst_ref, wedge_ref, aedge_ref,
               xw_ref, as_ref, ad_ref, s_ref):
    xw = jnp.dot(x_ref[...], w_ref[...], preferred_element_type=jnp.float32)
    xw_ref[...] = xw
    jj = lax.broadcasted_iota(jnp.int32, (D, H), 0) // C
    hh = lax.broadcasted_iota(jnp.int32, (D, H), 1)
    sel = (jj == hh).astype(jnp.float32)
    as_ref[...] = jnp.dot(xw * asrc_ref[...], sel,
                          preferred_element_type=jnp.float32)
    ad_ref[...] = jnp.dot(xw * adst_ref[...], sel,
                          preferred_element_type=jnp.float32)
    s_ref[...] = jnp.dot(wedge_ref[...] * aedge_ref[...], sel,
                         preferred_element_type=jnp.float32)


def _project(x, w, asrc_flat, adst_flat, wedge, aedge_flat, n):
    bn = 1000
    grid = (n // bn,)
    return pl.pallas_call(
        _proj_body,
        grid=grid,
        in_specs=[
            pl.BlockSpec((bn, D), lambda i: (i, 0)),
            pl.BlockSpec((D, D), lambda i: (0, 0)),
            pl.BlockSpec((1, D), lambda i: (0, 0)),
            pl.BlockSpec((1, D), lambda i: (0, 0)),
            pl.BlockSpec((1, D), lambda i: (0, 0)),
            pl.BlockSpec((1, D), lambda i: (0, 0)),
        ],
        out_specs=[
            pl.BlockSpec((bn, D), lambda i: (i, 0)),
            pl.BlockSpec((bn, H), lambda i: (i, 0)),
            pl.BlockSpec((bn, H), lambda i: (i, 0)),
            pl.BlockSpec((1, H), lambda i: (0, 0)),
        ],
        out_shape=[
            jax.ShapeDtypeStruct((n, D), jnp.float32),
            jax.ShapeDtypeStruct((n, H), jnp.float32),
            jax.ShapeDtypeStruct((n, H), jnp.float32),
            jax.ShapeDtypeStruct((1, H), jnp.float32),
        ],
    )(x, w, asrc_flat, adst_flat, wedge, aedge_flat)


def _make_sc_edge_pass(n, n_pad, e):
    total_chunks = -(-e // K)
    chunks_per_tile = -(-total_chunks // NW)
    chunks_per_tile += chunks_per_tile % 2
    rem = e % K
    e_floor = (e // K) * K
    rows_per_tile = n_pad // NS
    mesh = plsc.VectorSubcoreMesh(core_axis_name="c", subcore_axis_name="s")

    def body(ei_hbm, attr_hbm, asrc_hbm, adst_hbm, xw_hbm, svec_hbm,
             outp_hbm, denp_hbm,
             src_v, dst_v, attr_v, as_v, ad_v, xw_v, p_v, svec_v, stage_v,
             out_sh, den_sh, sem_ld, sem_g, sem_sc):
        cid = lax.axis_index("c")
        sid = lax.axis_index("s")
        wid = cid * NS + sid
        nch = chunks_per_tile

        pltpu.sync_copy(svec_hbm, svec_v)

        zeros16 = jnp.zeros((16,), jnp.float32)

        def zero_row(i, _):
            for j in range(D // 16):
                xw_v[0][i, pl.ds(j * 16, 16)] = zeros16
            p_v[0][i, pl.ds(0, 16)] = zeros16
            return 0

        lax.fori_loop(0, K, zero_row, 0)

        rbase = sid * rows_per_tile

        def zero_stripe(j, _):
            pltpu.sync_copy(xw_v[0].at[pl.ds(0, ZB)],
                            out_sh.at[pl.ds(rbase + j * ZB, ZB)])
            pltpu.sync_copy(p_v[0].at[pl.ds(0, ZB)],
                            den_sh.at[pl.ds(rbase + j * ZB, ZB)])
            return 0

        lax.fori_loop(0, rows_per_tile // ZB, zero_stripe, 0)
        plsc.subcore_barrier()

        lanes = lax.iota(jnp.int32, 16)
        ones16 = jnp.full((16,), 1.0, jnp.float32)
        col8 = jnp.full((16,), 8, jnp.int32)
        col9 = jnp.full((16,), 9, jnp.int32)

        def chunk_base(k):
            return (k * NW + wid) * K

        def load_idx(k, b):
            ebase = chunk_base(k)
            pltpu.async_copy(ei_hbm.at[pl.ds(ebase, K)], src_v[b],
                             sem_ld[0])
            pltpu.async_copy(ei_hbm.at[pl.ds(e + ebase, K)], dst_v[b],
                             sem_ld[0])
            pltpu.async_copy(attr_hbm.at[pl.ds(ebase, K)], attr_v[b],
                             sem_ld[0])

        def wait_idx(b):
            pltpu.make_async_copy(ei_hbm.at[pl.ds(0, K)], src_v[b],
                                  sem_ld[0]).wait()
            pltpu.make_async_copy(ei_hbm.at[pl.ds(0, K)], dst_v[b],
                                  sem_ld[0]).wait()
            pltpu.make_async_copy(attr_hbm.at[pl.ds(0, K)], attr_v[b],
                                  sem_ld[0]).wait()

        def load_stage(k):
            ebase = chunk_base(k)

            @pl.when(ebase + K <= e)
            def _full():
                pltpu.async_copy(ei_hbm.at[pl.ds(ebase, K)], stage_v[0],
                                 sem_ld[1])
                pltpu.async_copy(ei_hbm.at[pl.ds(e + ebase, K)], stage_v[1],
                                 sem_ld[1])
                pltpu.async_copy(attr_hbm.at[pl.ds(ebase, K)], stage_v[2],
                                 sem_ld[1])

            if rem:
                @pl.when(ebase == e_floor)
                def _partial():
                    pltpu.async_copy(ei_hbm.at[pl.ds(ebase, rem)],
                                     stage_v[0].at[pl.ds(0, rem)], sem_ld[1])
                    pltpu.async_copy(ei_hbm.at[pl.ds(e + ebase, rem)],
                                     stage_v[1].at[pl.ds(0, rem)], sem_ld[1])
                    pltpu.async_copy(attr_hbm.at[pl.ds(ebase, rem)],
                                     stage_v[2].at[pl.ds(0, rem)], sem_ld[1])

        def unstage(b, k):
            ebase = chunk_base(k)

            @pl.when(ebase + K <= e)
            def _full():
                pltpu.make_async_copy(ei_hbm.at[pl.ds(0, K)], stage_v[0],
                                      sem_ld[1]).wait()
                pltpu.make_async_copy(ei_hbm.at[pl.ds(0, K)], stage_v[1],
                                      sem_ld[1]).wait()
                pltpu.make_async_copy(attr_hbm.at[pl.ds(0, K)], stage_v[2],
                                      sem_ld[1]).wait()

            if rem:
                @pl.when(ebase == e_floor)
                def _partial():
                    pltpu.make_async_copy(ei_hbm.at[pl.ds(0, rem)],
                                          stage_v[0].at[pl.ds(0, rem)],
                                          sem_ld[1]).wait()
                    pltpu.make_async_copy(ei_hbm.at[pl.ds(0, rem)],
                                          stage_v[1].at[pl.ds(0, rem)],
                                          sem_ld[1]).wait()
                    pltpu.make_async_copy(attr_hbm.at[pl.ds(0, rem)],
                                          stage_v[2].at[pl.ds(0, rem)],
                                          sem_ld[1]).wait()

            @plsc.parallel_loop(0, K // 16, 1, unroll=2)
            def cp(g):
                sl = pl.ds(g * 16, 16)
                src_v[b][sl] = stage_v[0][sl]
                dst_v[b][sl] = stage_v[1][sl]
                attr_v[b][sl] = stage_v[2][sl]

        def start_gathers(b):
            pltpu.async_copy(asrc_hbm.at[src_v[b]], as_v[b], sem_g[b])
            pltpu.async_copy(adst_hbm.at[dst_v[b]], ad_v[b], sem_g[b])
            pltpu.async_copy(xw_hbm.at[src_v[b]], xw_v[b], sem_g[b])

        def wait_scatters(b, k):
            @pl.when(chunk_base(k) < e)
            def _():
                pltpu.make_async_copy(xw_v[b], out_sh.at[dst_v[b]],
                                      sem_sc[b]).wait()
                pltpu.make_async_copy(p_v[b], den_sh.at[dst_v[b]],
                                      sem_sc[b]).wait()

        def process(k, b):
            nb = 1 - b
            @pl.when(k + 1 < nch)
            def _prefetch():
                @pl.when(k >= 1)
                def _drain():
                    wait_scatters(nb, k - 1)

                unstage(nb, k + 1)

                @pl.when(chunk_base(k + 1) < e)
                def _gather():
                    start_gathers(nb)

            @pl.when(k + 2 < nch)
            def _stage():
                load_stage(k + 2)

            ebase = chunk_base(k)

            @pl.when(ebase < e)
            def _active():
                pltpu.make_async_copy(asrc_hbm.at[pl.ds(0, K)], as_v[b],
                                      sem_g[b]).wait()
                pltpu.make_async_copy(adst_hbm.at[pl.ds(0, K)], ad_v[b],
                                      sem_g[b]).wait()

                @plsc.parallel_loop(0, K // 16, 1, unroll=2)
                def grp(g):
                    eids = lanes + g * 16
                    valid = (ebase + eids) < e
                    att16 = attr_v[b][pl.ds(g * 16, 16)]
                    s_all = svec_v[pl.ds(0, 16)]
                    for h in range(H):
                        hv = jnp.full((16,), h, jnp.int32)
                        a_s = plsc.load_gather(as_v[b], [eids, hv])
                        a_d = plsc.load_gather(ad_v[b], [eids, hv])
                        al = a_s + a_d + att16 * s_all[h]
                        al = jnp.where(al >= 0.0, al, al * 0.2)
                        p = jnp.where(valid, jnp.exp(al), 0.0)
                        plsc.store_scatter(p_v[b], [eids, hv], p)
                    plsc.store_scatter(p_v[b], [eids, col8],
                                       jnp.where(valid, ones16, 0.0))
                    plsc.store_scatter(p_v[b], [eids, col9], att16)

                pltpu.make_async_copy(xw_hbm.at[pl.ds(0, K)], xw_v[b],
                                      sem_g[b]).wait()

                @plsc.parallel_loop(0, K, 1, unroll=4)
                def edge_mul(i):
                    prow = p_v[b][i, pl.ds(0, 16)]
                    for h in range(H):
                        xw_v[b][i, pl.ds(h * 16, 16)] = (
                            xw_v[b][i, pl.ds(h * 16, 16)] * prow[h])

                pltpu.async_copy(xw_v[b], out_sh.at[dst_v[b]], sem_sc[b],
                                 add=True)
                pltpu.async_copy(p_v[b], den_sh.at[dst_v[b]], sem_sc[b],
                                 add=True)

        load_idx(0, 0)
        wait_idx(0)
        start_gathers(0)
        load_stage(1)

        def pair_body(j, _):
            process(2 * j, 0)
            process(2 * j + 1, 1)
            return 0

        lax.fori_loop(0, nch // 2, pair_body, 0)
        wait_scatters(0, nch - 2)
        wait_scatters(1, nch - 1)
        plsc.subcore_barrier()

        pltpu.sync_copy(out_sh.at[pl.ds(rbase, rows_per_tile)],
                        outp_hbm.at[cid, pl.ds(rbase, rows_per_tile)])
        pltpu.sync_copy(den_sh.at[pl.ds(rbase, rows_per_tile)],
                        denp_hbm.at[cid, pl.ds(rbase, rows_per_tile)])

    return pl.kernel(
        body,
        out_type=[
            jax.ShapeDtypeStruct((NC, n_pad, D), jnp.float32),
            jax.ShapeDtypeStruct((NC, n_pad, 16), jnp.float32),
        ],
        mesh=mesh,
        compiler_params=pltpu.CompilerParams(
            needs_layout_passes=False, use_tc_tiling_on_sc=False),
        scratch_types=[
            (pltpu.VMEM((K,), jnp.int32), pltpu.VMEM((K,), jnp.int32)),
            (pltpu.VMEM((K,), jnp.int32), pltpu.VMEM((K,), jnp.int32)),
            (pltpu.VMEM((K,), jnp.float32), pltpu.VMEM((K,), jnp.float32)),
            (pltpu.VMEM((K, H), jnp.float32), pltpu.VMEM((K, H), jnp.float32)),
            (pltpu.VMEM((K, H), jnp.float32), pltpu.VMEM((K, H), jnp.float32)),
            (pltpu.VMEM((K, D), jnp.float32), pltpu.VMEM((K, D), jnp.float32)),
            (pltpu.VMEM((K, 16), jnp.float32), pltpu.VMEM((K, 16), jnp.float32)),
            pltpu.VMEM((16,), jnp.float32),
            (pltpu.VMEM((K,), jnp.int32), pltpu.VMEM((K,), jnp.int32),
             pltpu.VMEM((K,), jnp.float32)),
            pltpu.VMEM_SHARED((n_pad, D), jnp.float32),
            pltpu.VMEM_SHARED((n_pad, 16), jnp.float32),
            (pltpu.SemaphoreType.DMA, pltpu.SemaphoreType.DMA),
            (pltpu.SemaphoreType.DMA, pltpu.SemaphoreType.DMA),
            (pltpu.SemaphoreType.DMA, pltpu.SemaphoreType.DMA),
        ],
    )


def _final_body(outp_ref, denp_ref, xw_ref, as_ref, ad_ref, s_ref,
                bias_ref, gamma_ref, beta_ref, out_ref):
    den = denp_ref[0] + denp_ref[1]
    acc = outp_ref[0] + outp_ref[1]
    denh = den[:, 0:H]
    deg = den[:, H:H + 1]
    asum = den[:, H + 1:H + 2]
    loop_attr = asum / jnp.maximum(deg, 1.0)
    al = as_ref[...] + ad_ref[...] + loop_attr * s_ref[...]
    al = jnp.where(al >= 0.0, al, al * 0.2)
    p_loop = jnp.exp(al)
    denom = denh + p_loop
    hh = lax.broadcasted_iota(jnp.int32, (H, D), 0)
    jj = lax.broadcasted_iota(jnp.int32, (H, D), 1) // C
    sel = (hh == jj).astype(jnp.float32)
    acc = acc + jnp.dot(p_loop, sel, preferred_element_type=jnp.float32) * xw_ref[...]
    acc = acc / (jnp.dot(denom, sel, preferred_element_type=jnp.float32) + 1e-16)
    acc = (acc + bias_ref[...]) * (gamma_ref[...] * (1.0 / jnp.sqrt(1.0 + 1e-5))) + beta_ref[...]
    out_ref[...] = jnp.where(acc > 0.0, acc, jnp.exp(acc) - 1.0)


def _finalize(outp, denp, xw, a_src, a_dst, s_row, bias, gamma, beta, n):
    bn = 1000
    grid = (n // bn,)
    return pl.pallas_call(
        _final_body,
        grid=grid,
        in_specs=[
            pl.BlockSpec((NC, bn, D), lambda i: (0, i, 0)),
            pl.BlockSpec((NC, bn, 16), lambda i: (0, i, 0)),
            pl.BlockSpec((bn, D), lambda i: (i, 0)),
            pl.BlockSpec((bn, H), lambda i: (i, 0)),
            pl.BlockSpec((bn, H), lambda i: (i, 0)),
            pl.BlockSpec((1, H), lambda i: (0, 0)),
            pl.BlockSpec((1, D), lambda i: (0, 0)),
            pl.BlockSpec((1, D), lambda i: (0, 0)),
            pl.BlockSpec((1, D), lambda i: (0, 0)),
        ],
        out_specs=pl.BlockSpec((bn, D), lambda i: (i, 0)),
        out_shape=jax.ShapeDtypeStruct((n, D), jnp.float32),
    )(outp, denp, xw, a_src, a_dst, s_row, bias, gamma, beta)


@jax.jit
def kernel(x, edge_index, edge_attr, W, att_src, att_dst, W_edge, att_edge,
           bias, bn_gamma, bn_beta):
    n = x.shape[0]
    e = edge_index.shape[1]

    ei = edge_index.astype(jnp.int32)

    asrc_flat = att_src.reshape(1, D)
    adst_flat = att_dst.reshape(1, D)
    aedge_flat = att_edge.reshape(1, D)

    xw, a_src, a_dst, s_row = _project(x, W, asrc_flat, adst_flat,
                                       W_edge, aedge_flat, n)

    s16 = jnp.concatenate([s_row.reshape(H), jnp.zeros((16 - H,), jnp.float32)])
    n_pad = ((n + NS * ZB - 1) // (NS * ZB)) * (NS * ZB)
    outp, denp = _make_sc_edge_pass(n, n_pad, e)(
        ei.reshape(2 * e), edge_attr.reshape(e), a_src, a_dst, xw, s16)

    return _finalize(outp, denp, xw, a_src, a_dst, s_row,
                     bias.reshape(1, D), bn_gamma.reshape(1, D),
                     bn_beta.reshape(1, D), n)
